# baseline (speedup 1.0000x reference)
import functools

import jax
import jax.numpy as jnp
from jax import lax
from jax.experimental import pallas as pl
from jax.experimental.pallas import tpu as pltpu
from jax.experimental.pallas import tpu_sc as plsc

B, T, D_IN = 64, 8192, 32
HID = 32
LIFT_K = 16
D_MODEL = 128
K_SEL = 8
LAM = 0.5
K_EFF = 16

BBLK = 8
TBLK = 2048


def _score_body(x_ref, w1_ref, b1_ref, w2_ref, b2_ref, out_ref):
    xb = x_ref[...]
    xr = xb.reshape(BBLK * TBLK, D_IN)
    h = jnp.tanh(
        jnp.dot(xr, w1_ref[...], preferred_element_type=jnp.float32)
        + b1_ref[...]
    )
    s = jnp.dot(h, w2_ref[...], preferred_element_type=jnp.float32) + b2_ref[0, 0]
    sal = jax.nn.softplus(s)
    out_ref[...] = sal.reshape(BBLK, TBLK)


def _select_body(sal_ref, logt_ref, y_ref, idx_ref, ssal_ref, stn_ref, scum_ref):
    sal = sal_ref[...]
    temp = jnp.clip(jnp.exp(logt_ref[0, 0]), 0.1, 10.0)
    logits = (sal / (2.0 * LAM) - 0.5) / temp
    y = jax.nn.sigmoid(logits)
    tt = lax.broadcasted_iota(jnp.int32, (B, T), 1)
    y = jnp.where(tt == 0, 0.0, y)
    budget = jnp.maximum(jnp.sum(y, axis=1, keepdims=True), 1e-6)
    scale = jnp.minimum(K_SEL / budget, 1.0)
    y = y * scale
    shift = jnp.concatenate([y[:, 1:], y[:, :1]], axis=1)
    pair = y + shift
    damping = jnp.minimum(2.0 / (1.0 + pair), 1.0)
    y = y * damping
    y = jnp.where(tt == 0, 0.0, y)
    y_ref[...] = y

    total = jnp.sum(sal, axis=1, keepdims=True)
    denom = total + 1e-6
    bb = lax.broadcasted_iota(jnp.int32, (B, 1), 0)

    val = y
    idx_cols, ssal_cols, stn_cols, scum_cols = [], [], [], []
    for _ in range(K_EFF):
        m = jnp.max(val, axis=1, keepdims=True)
        cand = jnp.where(val == m, tt, T)
        idx = jnp.min(cand, axis=1, keepdims=True)
        oh = tt == idx
        ssal_cols.append(jnp.sum(jnp.where(oh, sal, 0.0), axis=1, keepdims=True))
        scum_cols.append(
            jnp.sum(jnp.where(tt <= idx, sal, 0.0), axis=1, keepdims=True) / denom
        )
        stn_cols.append(idx.astype(jnp.float32) * (1.0 / T))
        idx_cols.append(idx + bb * T)
        val = jnp.where(oh, -1.0, val)
    idx_ref[...] = jnp.concatenate(idx_cols, axis=1)
    ssal_ref[...] = jnp.concatenate(ssal_cols, axis=1)
    stn_ref[...] = jnp.concatenate(stn_cols, axis=1)
    scum_ref[...] = jnp.concatenate(scum_cols, axis=1)


_NC, _NS = 2, 16
_NW = _NC * _NS
_ROWS = B * K_EFF
_RPW = _ROWS // _NW


@functools.lru_cache(maxsize=None)
def _build_sc_gather():
    mesh = plsc.VectorSubcoreMesh(core_axis_name="c", subcore_axis_name="s")

    @functools.partial(
        pl.kernel,
        mesh=mesh,
        compiler_params=pltpu.CompilerParams(use_tc_tiling_on_sc=False),
        out_type=jax.ShapeDtypeStruct((_ROWS, D_IN), jnp.float32),
        scratch_types=[
            pltpu.VMEM((_RPW,), jnp.int32),
            pltpu.VMEM((_RPW, D_IN), jnp.float32),
            pltpu.SemaphoreType.DMA,
        ],
    )
    def _sc_gather(table_hbm, idx_hbm, out_hbm, idx_v, rows_v, sem):
        wid = lax.axis_index("s") * _NC + lax.axis_index("c")
        base = wid * _RPW
        pltpu.sync_copy(idx_hbm.at[pl.ds(base, _RPW)], idx_v)
        pltpu.async_copy(table_hbm.at[idx_v], rows_v, sem).wait()
        pltpu.sync_copy(rows_v, out_hbm.at[pl.ds(base, _RPW)])

    return _sc_gather


def _project_body(xg_ref, ssal_ref, stn_ref, scum_ref, wlx_ref, wls_ref,
                  wlt_ref, wlc_ref, bl_ref, wp_ref, bp_ref, out_ref):
    xg = xg_ref[...]
    ssal = ssal_ref[...]
    stn = stn_ref[...]
    scum = scum_ref[...]
    q = (jnp.sum(xg * xg, axis=1, keepdims=True)
         + ssal * ssal + stn * stn + scum * scum)
    inv = 1.0 / (jnp.sqrt(q) + 1e-6)
    lin = (jnp.dot(xg, wlx_ref[...], preferred_element_type=jnp.float32)
           + ssal * wls_ref[...] + stn * wlt_ref[...] + scum * wlc_ref[...])
    lifted = jnp.tanh(lin * inv + bl_ref[...])
    out_ref[...] = (
        jnp.dot(lifted, wp_ref[...], preferred_element_type=jnp.float32)
        + bp_ref[...]
    )


def _tc_pipeline(x, W1, b1, w2, b2, log_temperature):
    sal = pl.pallas_call(
        _score_body,
        grid=(B // BBLK, T // TBLK),
        in_specs=[
            pl.BlockSpec((BBLK, TBLK, D_IN), lambda i, j: (i, j, 0)),
            pl.BlockSpec((D_IN, HID), lambda i, j: (0, 0)),
            pl.BlockSpec((1, HID), lambda i, j: (0, 0)),
            pl.BlockSpec((HID, 1), lambda i, j: (0, 0)),
            pl.BlockSpec(memory_space=pltpu.SMEM),
        ],
        out_specs=pl.BlockSpec((BBLK, TBLK), lambda i, j: (i, j)),
        out_shape=jax.ShapeDtypeStruct((B, T), jnp.float32),
    )(x, W1, b1.reshape(1, HID), w2.reshape(HID, 1), b2.reshape(1, 1))

    y, fidx, ssal, stn, scum = pl.pallas_call(
        _select_body,
        in_specs=[
            pl.BlockSpec((B, T), lambda: (0, 0)),
            pl.BlockSpec(memory_space=pltpu.SMEM),
        ],
        out_specs=[
            pl.BlockSpec((B, T), lambda: (0, 0)),
            pl.BlockSpec((B, K_EFF), lambda: (0, 0)),
            pl.BlockSpec((B, K_EFF), lambda: (0, 0)),
            pl.BlockSpec((B, K_EFF), lambda: (0, 0)),
            pl.BlockSpec((B, K_EFF), lambda: (0, 0)),
        ],
        out_shape=[
            jax.ShapeDtypeStruct((B, T), jnp.float32),
            jax.ShapeDtypeStruct((B, K_EFF), jnp.int32),
            jax.ShapeDtypeStruct((B, K_EFF), jnp.float32),
            jax.ShapeDtypeStruct((B, K_EFF), jnp.float32),
            jax.ShapeDtypeStruct((B, K_EFF), jnp.float32),
        ],
    )(sal, log_temperature.reshape(1, 1))
    return y, fidx, ssal, stn, scum


def _project(xg, ssal, stn, scum, W_lift, b_lift, Wp, bp):
    return pl.pallas_call(
        _project_body,
        in_specs=[
            pl.BlockSpec((_ROWS, D_IN), lambda: (0, 0)),
            pl.BlockSpec((_ROWS, 1), lambda: (0, 0)),
            pl.BlockSpec((_ROWS, 1), lambda: (0, 0)),
            pl.BlockSpec((_ROWS, 1), lambda: (0, 0)),
            pl.BlockSpec((D_IN, LIFT_K), lambda: (0, 0)),
            pl.BlockSpec((1, LIFT_K), lambda: (0, 0)),
            pl.BlockSpec((1, LIFT_K), lambda: (0, 0)),
            pl.BlockSpec((1, LIFT_K), lambda: (0, 0)),
            pl.BlockSpec((1, LIFT_K), lambda: (0, 0)),
            pl.BlockSpec((LIFT_K, D_MODEL), lambda: (0, 0)),
            pl.BlockSpec((1, D_MODEL), lambda: (0, 0)),
        ],
        out_specs=pl.BlockSpec((_ROWS, D_MODEL), lambda: (0, 0)),
        out_shape=jax.ShapeDtypeStruct((_ROWS, D_MODEL), jnp.float32),
    )(
        xg,
        ssal.reshape(_ROWS, 1), stn.reshape(_ROWS, 1), scum.reshape(_ROWS, 1),
        W_lift[:D_IN], W_lift[D_IN].reshape(1, LIFT_K),
        W_lift[D_IN + 1].reshape(1, LIFT_K), W_lift[D_IN + 2].reshape(1, LIFT_K),
        b_lift.reshape(1, LIFT_K), Wp, bp.reshape(1, D_MODEL),
    )


def kernel(x, W1, b1, w2, b2, log_temperature, W_lift, b_lift, Wp, bp):
    y, fidx, ssal, stn, scum = _tc_pipeline(x, W1, b1, w2, b2, log_temperature)
    xg = _build_sc_gather()(x.reshape(B * T, D_IN), fidx.reshape(_ROWS))
    tokens = _project(xg, ssal, stn, scum, W_lift, b_lift, Wp, bp)
    return tokens.reshape(B, K_EFF, D_MODEL), y


# speedup vs baseline: 3.0540x; 3.0540x over previous
import functools

import jax
import jax.numpy as jnp
from jax import lax
from jax.experimental import pallas as pl
from jax.experimental.pallas import tpu as pltpu
from jax.experimental.pallas import tpu_sc as plsc

B, T, D_IN = 64, 8192, 32
HID = 32
LIFT_K = 16
D_MODEL = 128
K_SEL = 8
LAM = 0.5
K_EFF = 16

NPC = 4
TP = T // NPC
PACKW = NPC * D_IN

BBLK = 2


def _score_body(xt_ref, w1t_ref, b1_ref, w2_ref, b2_ref, sal_ref, pack_ref):
    w1t = w1t_ref[...]
    b1c = b1_ref[...]
    w2r = w2_ref[...]
    for b in range(BBLK):
        xb = xt_ref[b]
        h = jnp.tanh(
            jnp.dot(w1t, xb, preferred_element_type=jnp.float32) + b1c
        )
        s = jnp.dot(w2r, h, preferred_element_type=jnp.float32) + b2_ref[0, 0]
        sal_ref[b] = jax.nn.softplus(s)
        pack_ref[b] = jnp.concatenate(
            [xb[:, p * TP:(p + 1) * TP].T for p in range(NPC)], axis=1
        )


def _select_body(sal_ref, logt_ref, y_ref, idx_ref, fp_ref,
                 ssal_ref, stn_ref, scum_ref):
    sal = sal_ref[...]
    temp = jnp.clip(jnp.exp(logt_ref[0, 0]), 0.1, 10.0)
    logits = (sal / (2.0 * LAM) - 0.5) / temp
    y = jax.nn.sigmoid(logits)
    tt = lax.broadcasted_iota(jnp.int32, (B, T), 1)
    y = jnp.where(tt == 0, 0.0, y)
    budget = jnp.maximum(jnp.sum(y, axis=1, keepdims=True), 1e-6)
    scale = jnp.minimum(K_SEL / budget, 1.0)
    y = y * scale
    shift = jnp.concatenate([y[:, 1:], y[:, :1]], axis=1)
    pair = y + shift
    damping = jnp.minimum(2.0 / (1.0 + pair), 1.0)
    y = y * damping
    y = jnp.where(tt == 0, 0.0, y)
    y_ref[...] = y

    total = jnp.sum(sal, axis=1, keepdims=True)
    denom = total + 1e-6
    bb = lax.broadcasted_iota(jnp.int32, (B, 1), 0)

    val = y
    idx_cols, fp_cols, ssal_cols, stn_cols, scum_cols = [], [], [], [], []
    for _ in range(K_EFF):
        m = jnp.max(val, axis=1, keepdims=True)
        cand = jnp.where(val == m, tt, T)
        idx = jnp.min(cand, axis=1, keepdims=True)
        oh = tt == idx
        ssal_cols.append(jnp.sum(jnp.where(oh, sal, 0.0), axis=1, keepdims=True))
        scum_cols.append(
            jnp.sum(jnp.where(tt <= idx, sal, 0.0), axis=1, keepdims=True) / denom
        )
        stn_cols.append(idx.astype(jnp.float32) * (1.0 / T))
        idx_cols.append((idx & (TP - 1)) + bb * TP)
        fp_cols.append(idx >> 11)
        val = jnp.where(oh, -1.0, val)
    idx_ref[...] = jnp.concatenate(idx_cols, axis=1)
    fp_ref[...] = jnp.concatenate(fp_cols, axis=1)
    ssal_ref[...] = jnp.concatenate(ssal_cols, axis=1)
    stn_ref[...] = jnp.concatenate(stn_cols, axis=1)
    scum_ref[...] = jnp.concatenate(scum_cols, axis=1)


_NC, _NS = 2, 16
_NW = _NC * _NS
_ROWS = B * K_EFF
_RPW = _ROWS // _NW


@functools.lru_cache(maxsize=None)
def _build_sc_gather():
    mesh = plsc.VectorSubcoreMesh(core_axis_name="c", subcore_axis_name="s")

    @functools.partial(
        pl.kernel,
        mesh=mesh,
        compiler_params=pltpu.CompilerParams(use_tc_tiling_on_sc=False),
        out_type=jax.ShapeDtypeStruct((_ROWS, PACKW), jnp.float32),
        scratch_types=[
            pltpu.VMEM((_RPW,), jnp.int32),
            pltpu.VMEM((_RPW, PACKW), jnp.float32),
            pltpu.SemaphoreType.DMA,
        ],
    )
    def _sc_gather(table_hbm, idx_hbm, out_hbm, idx_v, rows_v, sem):
        wid = lax.axis_index("s") * _NC + lax.axis_index("c")
        base = wid * _RPW
        pltpu.sync_copy(idx_hbm.at[pl.ds(base, _RPW)], idx_v)
        pltpu.async_copy(table_hbm.at[idx_v], rows_v, sem).wait()
        pltpu.sync_copy(rows_v, out_hbm.at[pl.ds(base, _RPW)])

    return _sc_gather


def _project_body(xg_ref, fp_ref, ssal_ref, stn_ref, scum_ref, wlx_ref,
                  wls_ref, wlt_ref, wlc_ref, bl_ref, wp_ref, bp_ref, out_ref):
    xg = xg_ref[...]
    fp = fp_ref[...]
    xs = jnp.where(
        fp == 0, xg[:, 0:32],
        jnp.where(fp == 1, xg[:, 32:64],
                  jnp.where(fp == 2, xg[:, 64:96], xg[:, 96:128])))
    ssal = ssal_ref[...]
    stn = stn_ref[...]
    scum = scum_ref[...]
    q = (jnp.sum(xs * xs, axis=1, keepdims=True)
         + ssal * ssal + stn * stn + scum * scum)
    inv = 1.0 / (jnp.sqrt(q) + 1e-6)
    lin = (jnp.dot(xs, wlx_ref[...], preferred_element_type=jnp.float32)
           + ssal * wls_ref[...] + stn * wlt_ref[...] + scum * wlc_ref[...])
    lifted = jnp.tanh(lin * inv + bl_ref[...])
    out_ref[...] = (
        jnp.dot(lifted, wp_ref[...], preferred_element_type=jnp.float32)
        + bp_ref[...]
    )


def _tc_pipeline(x, W1, b1, w2, b2, log_temperature):
    xt = jnp.transpose(x, (0, 2, 1))
    sal3, xpack = pl.pallas_call(
        _score_body,
        grid=(B // BBLK,),
        in_specs=[
            pl.BlockSpec((BBLK, D_IN, T), lambda i: (i, 0, 0)),
            pl.BlockSpec((HID, D_IN), lambda i: (0, 0)),
            pl.BlockSpec((HID, 1), lambda i: (0, 0)),
            pl.BlockSpec((1, HID), lambda i: (0, 0)),
            pl.BlockSpec(memory_space=pltpu.SMEM),
        ],
        out_specs=[
            pl.BlockSpec((BBLK, 1, T), lambda i: (i, 0, 0)),
            pl.BlockSpec((BBLK, TP, PACKW), lambda i: (i, 0, 0)),
        ],
        out_shape=[
            jax.ShapeDtypeStruct((B, 1, T), jnp.float32),
            jax.ShapeDtypeStruct((B, TP, PACKW), jnp.float32),
        ],
    )(xt, W1.T, b1.reshape(HID, 1), w2.reshape(1, HID), b2.reshape(1, 1))

    y, fidx, fp, ssal, stn, scum = pl.pallas_call(
        _select_body,
        in_specs=[
            pl.BlockSpec((B, T), lambda: (0, 0)),
            pl.BlockSpec(memory_space=pltpu.SMEM),
        ],
        out_specs=[
            pl.BlockSpec((B, T), lambda: (0, 0)),
            pl.BlockSpec((B, K_EFF), lambda: (0, 0)),
            pl.BlockSpec((B, K_EFF), lambda: (0, 0)),
            pl.BlockSpec((B, K_EFF), lambda: (0, 0)),
            pl.BlockSpec((B, K_EFF), lambda: (0, 0)),
            pl.BlockSpec((B, K_EFF), lambda: (0, 0)),
        ],
        out_shape=[
            jax.ShapeDtypeStruct((B, T), jnp.float32),
            jax.ShapeDtypeStruct((B, K_EFF), jnp.int32),
            jax.ShapeDtypeStruct((B, K_EFF), jnp.int32),
            jax.ShapeDtypeStruct((B, K_EFF), jnp.float32),
            jax.ShapeDtypeStruct((B, K_EFF), jnp.float32),
            jax.ShapeDtypeStruct((B, K_EFF), jnp.float32),
        ],
    )(sal3.reshape(B, T), log_temperature.reshape(1, 1))
    return y, xpack, fidx, fp, ssal, stn, scum


def _project(xg, fp, ssal, stn, scum, W_lift, b_lift, Wp, bp):
    return pl.pallas_call(
        _project_body,
        in_specs=[
            pl.BlockSpec((_ROWS, PACKW), lambda: (0, 0)),
            pl.BlockSpec((_ROWS, 1), lambda: (0, 0)),
            pl.BlockSpec((_ROWS, 1), lambda: (0, 0)),
            pl.BlockSpec((_ROWS, 1), lambda: (0, 0)),
            pl.BlockSpec((_ROWS, 1), lambda: (0, 0)),
            pl.BlockSpec((D_IN, LIFT_K), lambda: (0, 0)),
            pl.BlockSpec((1, LIFT_K), lambda: (0, 0)),
            pl.BlockSpec((1, LIFT_K), lambda: (0, 0)),
            pl.BlockSpec((1, LIFT_K), lambda: (0, 0)),
            pl.BlockSpec((1, LIFT_K), lambda: (0, 0)),
            pl.BlockSpec((LIFT_K, D_MODEL), lambda: (0, 0)),
            pl.BlockSpec((1, D_MODEL), lambda: (0, 0)),
        ],
        out_specs=pl.BlockSpec((_ROWS, D_MODEL), lambda: (0, 0)),
        out_shape=jax.ShapeDtypeStruct((_ROWS, D_MODEL), jnp.float32),
    )(
        xg, fp.reshape(_ROWS, 1),
        ssal.reshape(_ROWS, 1), stn.reshape(_ROWS, 1), scum.reshape(_ROWS, 1),
        W_lift[:D_IN], W_lift[D_IN].reshape(1, LIFT_K),
        W_lift[D_IN + 1].reshape(1, LIFT_K), W_lift[D_IN + 2].reshape(1, LIFT_K),
        b_lift.reshape(1, LIFT_K), Wp, bp.reshape(1, D_MODEL),
    )


def kernel(x, W1, b1, w2, b2, log_temperature, W_lift, b_lift, Wp, bp):
    y, xpack, fidx, fp, ssal, stn, scum = _tc_pipeline(
        x, W1, b1, w2, b2, log_temperature)
    xg = _build_sc_gather()(xpack.reshape(B * TP, PACKW), fidx.reshape(_ROWS))
    tokens = _project(xg, fp, ssal, stn, scum, W_lift, b_lift, Wp, bp)
    return tokens.reshape(B, K_EFF, D_MODEL), y


# speedup vs baseline: 3.0824x; 1.0093x over previous
import functools

import jax
import jax.numpy as jnp
from jax import lax
from jax.experimental import pallas as pl
from jax.experimental.pallas import tpu as pltpu
from jax.experimental.pallas import tpu_sc as plsc

B, T, D_IN = 64, 8192, 32
HID = 32
LIFT_K = 16
D_MODEL = 128
K_SEL = 8
LAM = 0.5
K_EFF = 16

NPC = 4
TP = T // NPC
PACKW = NPC * D_IN

BBLK = 8


def _score_body(xt_ref, w1t_ref, b1_ref, w2_ref, b2_ref, sal_ref, pack_ref):
    w1t = w1t_ref[...]
    b1c = b1_ref[...]
    w2r = w2_ref[...]
    sal_rows = []
    for b in range(BBLK):
        xb = xt_ref[b]
        h = jnp.tanh(
            jnp.dot(w1t, xb, preferred_element_type=jnp.float32) + b1c
        )
        s = jnp.dot(w2r, h, preferred_element_type=jnp.float32) + b2_ref[0, 0]
        sal_rows.append(jax.nn.softplus(s))
        for p in range(NPC):
            pack_ref[b, :, p * D_IN:(p + 1) * D_IN] = xb[:, p * TP:(p + 1) * TP].T
    sal_ref[...] = jnp.concatenate(sal_rows, axis=0)


def _select_body(sal_ref, logt_ref, y_ref, idx_ref, fp_ref,
                 ssal_ref, stn_ref, scum_ref):
    sal = sal_ref[...]
    temp = jnp.clip(jnp.exp(logt_ref[0, 0]), 0.1, 10.0)
    logits = (sal / (2.0 * LAM) - 0.5) / temp
    y = jax.nn.sigmoid(logits)
    tt = lax.broadcasted_iota(jnp.int32, (B, T), 1)
    y = jnp.where(tt == 0, 0.0, y)
    budget = jnp.maximum(jnp.sum(y, axis=1, keepdims=True), 1e-6)
    scale = jnp.minimum(K_SEL / budget, 1.0)
    y = y * scale
    shift = jnp.concatenate([y[:, 1:], y[:, :1]], axis=1)
    pair = y + shift
    damping = jnp.minimum(2.0 / (1.0 + pair), 1.0)
    y = y * damping
    y = jnp.where(tt == 0, 0.0, y)
    y_ref[...] = y

    total = jnp.sum(sal, axis=1, keepdims=True)
    denom = total + 1e-6
    bb = lax.broadcasted_iota(jnp.int32, (B, 1), 0)

    val = y
    idx_cols, fp_cols, ssal_cols, stn_cols, scum_cols = [], [], [], [], []
    for _ in range(K_EFF):
        m = jnp.max(val, axis=1, keepdims=True)
        cand = jnp.where(val == m, tt, T)
        idx = jnp.min(cand, axis=1, keepdims=True)
        oh = tt == idx
        ssal_cols.append(jnp.sum(jnp.where(oh, sal, 0.0), axis=1, keepdims=True))
        scum_cols.append(
            jnp.sum(jnp.where(tt <= idx, sal, 0.0), axis=1, keepdims=True) / denom
        )
        stn_cols.append(idx.astype(jnp.float32) * (1.0 / T))
        idx_cols.append((idx & (TP - 1)) + bb * TP)
        fp_cols.append(idx >> 11)
        val = jnp.where(oh, -1.0, val)
    idx_ref[...] = jnp.concatenate(idx_cols, axis=1)
    fp_ref[...] = jnp.concatenate(fp_cols, axis=1)
    ssal_ref[...] = jnp.concatenate(ssal_cols, axis=1)
    stn_ref[...] = jnp.concatenate(stn_cols, axis=1)
    scum_ref[...] = jnp.concatenate(scum_cols, axis=1)


_NC, _NS = 2, 16
_NW = _NC * _NS
_ROWS = B * K_EFF
_RPW = _ROWS // _NW


@functools.lru_cache(maxsize=None)
def _build_sc_gather():
    mesh = plsc.VectorSubcoreMesh(core_axis_name="c", subcore_axis_name="s")

    @functools.partial(
        pl.kernel,
        mesh=mesh,
        compiler_params=pltpu.CompilerParams(use_tc_tiling_on_sc=False),
        out_type=jax.ShapeDtypeStruct((_ROWS, PACKW), jnp.float32),
        scratch_types=[
            pltpu.VMEM((_RPW,), jnp.int32),
            pltpu.VMEM((_RPW, PACKW), jnp.float32),
            pltpu.SemaphoreType.DMA,
        ],
    )
    def _sc_gather(table_hbm, idx_hbm, out_hbm, idx_v, rows_v, sem):
        wid = lax.axis_index("s") * _NC + lax.axis_index("c")
        base = wid * _RPW
        pltpu.sync_copy(idx_hbm.at[pl.ds(base, _RPW)], idx_v)
        pltpu.async_copy(table_hbm.at[idx_v], rows_v, sem).wait()
        pltpu.sync_copy(rows_v, out_hbm.at[pl.ds(base, _RPW)])

    return _sc_gather


def _project_body(xg_ref, fp_ref, ssal_ref, stn_ref, scum_ref, wlx_ref,
                  wls_ref, wlt_ref, wlc_ref, bl_ref, wp_ref, bp_ref, out_ref):
    xg = xg_ref[...]
    fp = fp_ref[...]
    xs = jnp.where(
        fp == 0, xg[:, 0:32],
        jnp.where(fp == 1, xg[:, 32:64],
                  jnp.where(fp == 2, xg[:, 64:96], xg[:, 96:128])))
    ssal = ssal_ref[...]
    stn = stn_ref[...]
    scum = scum_ref[...]
    q = (jnp.sum(xs * xs, axis=1, keepdims=True)
         + ssal * ssal + stn * stn + scum * scum)
    inv = 1.0 / (jnp.sqrt(q) + 1e-6)
    lin = (jnp.dot(xs, wlx_ref[...], preferred_element_type=jnp.float32)
           + ssal * wls_ref[...] + stn * wlt_ref[...] + scum * wlc_ref[...])
    lifted = jnp.tanh(lin * inv + bl_ref[...])
    out_ref[...] = (
        jnp.dot(lifted, wp_ref[...], preferred_element_type=jnp.float32)
        + bp_ref[...]
    )


def _tc_pipeline(x, W1, b1, w2, b2, log_temperature):
    xt = jnp.transpose(x, (0, 2, 1))
    sal3, xpack = pl.pallas_call(
        _score_body,
        grid=(B // BBLK,),
        in_specs=[
            pl.BlockSpec((BBLK, D_IN, T), lambda i: (i, 0, 0)),
            pl.BlockSpec((HID, D_IN), lambda i: (0, 0)),
            pl.BlockSpec((HID, 1), lambda i: (0, 0)),
            pl.BlockSpec((1, HID), lambda i: (0, 0)),
            pl.BlockSpec(memory_space=pltpu.SMEM),
        ],
        out_specs=[
            pl.BlockSpec((BBLK, T), lambda i: (i, 0)),
            pl.BlockSpec((BBLK, TP, PACKW), lambda i: (i, 0, 0)),
        ],
        out_shape=[
            jax.ShapeDtypeStruct((B, T), jnp.float32),
            jax.ShapeDtypeStruct((B, TP, PACKW), jnp.float32),
        ],
    )(xt, W1.T, b1.reshape(HID, 1), w2.reshape(1, HID), b2.reshape(1, 1))

    y, fidx, fp, ssal, stn, scum = pl.pallas_call(
        _select_body,
        in_specs=[
            pl.BlockSpec((B, T), lambda: (0, 0)),
            pl.BlockSpec(memory_space=pltpu.SMEM),
        ],
        out_specs=[
            pl.BlockSpec((B, T), lambda: (0, 0)),
            pl.BlockSpec((B, K_EFF), lambda: (0, 0)),
            pl.BlockSpec((B, K_EFF), lambda: (0, 0)),
            pl.BlockSpec((B, K_EFF), lambda: (0, 0)),
            pl.BlockSpec((B, K_EFF), lambda: (0, 0)),
            pl.BlockSpec((B, K_EFF), lambda: (0, 0)),
        ],
        out_shape=[
            jax.ShapeDtypeStruct((B, T), jnp.float32),
            jax.ShapeDtypeStruct((B, K_EFF), jnp.int32),
            jax.ShapeDtypeStruct((B, K_EFF), jnp.int32),
            jax.ShapeDtypeStruct((B, K_EFF), jnp.float32),
            jax.ShapeDtypeStruct((B, K_EFF), jnp.float32),
            jax.ShapeDtypeStruct((B, K_EFF), jnp.float32),
        ],
    )(sal3, log_temperature.reshape(1, 1))
    return y, xpack, fidx, fp, ssal, stn, scum


def _project(xg, fp, ssal, stn, scum, W_lift, b_lift, Wp, bp):
    return pl.pallas_call(
        _project_body,
        in_specs=[
            pl.BlockSpec((_ROWS, PACKW), lambda: (0, 0)),
            pl.BlockSpec((_ROWS, 1), lambda: (0, 0)),
            pl.BlockSpec((_ROWS, 1), lambda: (0, 0)),
            pl.BlockSpec((_ROWS, 1), lambda: (0, 0)),
            pl.BlockSpec((_ROWS, 1), lambda: (0, 0)),
            pl.BlockSpec((D_IN, LIFT_K), lambda: (0, 0)),
            pl.BlockSpec((1, LIFT_K), lambda: (0, 0)),
            pl.BlockSpec((1, LIFT_K), lambda: (0, 0)),
            pl.BlockSpec((1, LIFT_K), lambda: (0, 0)),
            pl.BlockSpec((1, LIFT_K), lambda: (0, 0)),
            pl.BlockSpec((LIFT_K, D_MODEL), lambda: (0, 0)),
            pl.BlockSpec((1, D_MODEL), lambda: (0, 0)),
        ],
        out_specs=pl.BlockSpec((_ROWS, D_MODEL), lambda: (0, 0)),
        out_shape=jax.ShapeDtypeStruct((_ROWS, D_MODEL), jnp.float32),
    )(
        xg, fp.reshape(_ROWS, 1),
        ssal.reshape(_ROWS, 1), stn.reshape(_ROWS, 1), scum.reshape(_ROWS, 1),
        W_lift[:D_IN], W_lift[D_IN].reshape(1, LIFT_K),
        W_lift[D_IN + 1].reshape(1, LIFT_K), W_lift[D_IN + 2].reshape(1, LIFT_K),
        b_lift.reshape(1, LIFT_K), Wp, bp.reshape(1, D_MODEL),
    )


def kernel(x, W1, b1, w2, b2, log_temperature, W_lift, b_lift, Wp, bp):
    y, xpack, fidx, fp, ssal, stn, scum = _tc_pipeline(
        x, W1, b1, w2, b2, log_temperature)
    xg = _build_sc_gather()(xpack.reshape(B * TP, PACKW), fidx.reshape(_ROWS))
    tokens = _project(xg, fp, ssal, stn, scum, W_lift, b_lift, Wp, bp)
    return tokens.reshape(B, K_EFF, D_MODEL), y
